# baseline (speedup 1.0000x reference)
import functools

import jax
import jax.numpy as jnp
from jax.experimental import pallas as pl

_TOP_K = 8


def _router_block(x_ref, w1_ref, b1_ref, w2_ref, b2_ref,
                  idx_ref, wts_ref, logits_ref):
    x = x_ref[...]
    h = jnp.dot(x, w1_ref[...], preferred_element_type=jnp.float32)
    h = jnp.maximum(h + b1_ref[...], 0.0)
    logits = jnp.dot(h, w2_ref[...], preferred_element_type=jnp.float32)
    logits = logits + b2_ref[...]
    logits_ref[...] = logits

    t, e = logits.shape
    iota = jax.lax.broadcasted_iota(jnp.int32, (t, e), 1)
    cur = logits
    vals = []
    idxs = []
    for _ in range(_TOP_K):
        m = jnp.max(cur, axis=-1, keepdims=True)
        idx = jnp.min(jnp.where(cur == m, iota, e), axis=-1, keepdims=True)
        vals.append(m)
        idxs.append(idx)
        cur = jnp.where(iota == idx, -jnp.inf, cur)
    vals = jnp.concatenate(vals, axis=-1)
    idxs = jnp.concatenate(idxs, axis=-1)
    w = jnp.exp(vals - vals[:, :1])
    w = w / jnp.sum(w, axis=-1, keepdims=True)
    idx_ref[...] = idxs
    wts_ref[...] = w


@functools.partial(jax.jit, static_argnames=("block_t",))
def _router(x, W1, b1, W2, b2, block_t=512):
    n, d = x.shape
    hdim = W1.shape[1]
    e = W2.shape[1]
    grid = (n // block_t,)
    out = pl.pallas_call(
        _router_block,
        grid=grid,
        in_specs=[
            pl.BlockSpec((block_t, d), lambda i: (i, 0)),
            pl.BlockSpec((d, hdim), lambda i: (0, 0)),
            pl.BlockSpec((1, hdim), lambda i: (0, 0)),
            pl.BlockSpec((hdim, e), lambda i: (0, 0)),
            pl.BlockSpec((1, e), lambda i: (0, 0)),
        ],
        out_specs=[
            pl.BlockSpec((block_t, _TOP_K), lambda i: (i, 0)),
            pl.BlockSpec((block_t, _TOP_K), lambda i: (i, 0)),
            pl.BlockSpec((block_t, e), lambda i: (i, 0)),
        ],
        out_shape=[
            jax.ShapeDtypeStruct((n, _TOP_K), jnp.int32),
            jax.ShapeDtypeStruct((n, _TOP_K), jnp.float32),
            jax.ShapeDtypeStruct((n, e), jnp.float32),
        ],
    )(x, W1, b1.reshape(1, hdim), W2, b2.reshape(1, e))
    return out[0], out[1], out[2]


def kernel(x, W1, b1, W2, b2):
    return _router(x, W1, b1, W2, b2)


# speedup vs baseline: 1.0567x; 1.0567x over previous
import functools

import jax
import jax.numpy as jnp
from jax.experimental import pallas as pl
from jax.experimental.pallas import tpu as pltpu

_TOP_K = 8


def _router_block(x_ref, w1_ref, b1_ref, w2_ref, b2_ref,
                  idx_ref, wts_ref, logits_ref, scratch_ref):
    i = pl.program_id(0)
    nsteps = pl.num_programs(0)
    slot = jax.lax.rem(i, 2)

    @pl.when(i < nsteps - 1)
    def _matmul():
        x = x_ref[...]
        h = jnp.dot(x, w1_ref[...], preferred_element_type=jnp.float32)
        h = jnp.maximum(h + b1_ref[...], 0.0)
        lg = jnp.dot(h, w2_ref[...], preferred_element_type=jnp.float32)
        lg = lg + b2_ref[...]
        logits_ref[...] = lg
        scratch_ref[slot] = lg

    @pl.when(i > 0)
    def _topk():
        logits = scratch_ref[1 - slot]
        t, e = logits.shape
        iota = jax.lax.broadcasted_iota(jnp.int32, (t, e), 1)
        cur = logits
        vals = []
        idxs = []
        for _ in range(_TOP_K):
            m = jnp.max(cur, axis=-1, keepdims=True)
            idx = jnp.min(jnp.where(cur == m, iota, e), axis=-1, keepdims=True)
            vals.append(m)
            idxs.append(idx)
            cur = jnp.where(iota == idx, -jnp.inf, cur)
        vals = jnp.concatenate(vals, axis=-1)
        idxs = jnp.concatenate(idxs, axis=-1)
        w = jnp.exp(vals - vals[:, :1])
        w = w / jnp.sum(w, axis=-1, keepdims=True)
        idx_ref[...] = idxs
        wts_ref[...] = w


@functools.partial(jax.jit, static_argnames=("block_t",))
def _router(x, W1, b1, W2, b2, block_t=512):
    n, d = x.shape
    hdim = W1.shape[1]
    e = W2.shape[1]
    nblk = n // block_t
    last = nblk - 1
    grid = (nblk + 1,)
    out = pl.pallas_call(
        _router_block,
        grid=grid,
        in_specs=[
            pl.BlockSpec((block_t, d), lambda i: (jnp.minimum(i, last), 0)),
            pl.BlockSpec((d, hdim), lambda i: (0, 0)),
            pl.BlockSpec((1, hdim), lambda i: (0, 0)),
            pl.BlockSpec((hdim, e), lambda i: (0, 0)),
            pl.BlockSpec((1, e), lambda i: (0, 0)),
        ],
        out_specs=[
            pl.BlockSpec((block_t, _TOP_K), lambda i: (jnp.maximum(i - 1, 0), 0)),
            pl.BlockSpec((block_t, _TOP_K), lambda i: (jnp.maximum(i - 1, 0), 0)),
            pl.BlockSpec((block_t, e), lambda i: (jnp.minimum(i, last), 0)),
        ],
        out_shape=[
            jax.ShapeDtypeStruct((n, _TOP_K), jnp.int32),
            jax.ShapeDtypeStruct((n, _TOP_K), jnp.float32),
            jax.ShapeDtypeStruct((n, e), jnp.float32),
        ],
        scratch_shapes=[pltpu.VMEM((2, block_t, e), jnp.float32)],
    )(x, W1, b1.reshape(1, hdim), W2, b2.reshape(1, e))
    return out[0], out[1], out[2]


def kernel(x, W1, b1, W2, b2):
    return _router(x, W1, b1, W2, b2)


# speedup vs baseline: 1.1657x; 1.1031x over previous
import functools

import jax
import jax.numpy as jnp
from jax.experimental import pallas as pl
from jax.experimental.pallas import tpu as pltpu

_TOP_K = 8


def _router_block(x_ref, w1_ref, b1_ref, w2_ref, b2_ref,
                  idx_ref, wts_ref, logits_ref, scratch_ref):
    i = pl.program_id(0)
    nsteps = pl.num_programs(0)
    slot = jax.lax.rem(i, 2)

    @pl.when(i < nsteps - 1)
    def _matmul():
        x = x_ref[...]
        h = jnp.dot(x, w1_ref[...], preferred_element_type=jnp.float32)
        h = jnp.maximum(h + b1_ref[...], 0.0)
        lg = jnp.dot(h, w2_ref[...], preferred_element_type=jnp.float32)
        lg = lg + b2_ref[...]
        logits_ref[...] = lg
        scratch_ref[slot] = lg

    @pl.when(i > 0)
    def _topk():
        logits = scratch_ref[1 - slot]
        t, e = logits.shape
        iota = jax.lax.broadcasted_iota(jnp.int32, (t, e), 1)
        cur = logits
        vals = []
        idxs = []
        for _ in range(_TOP_K):
            m = jnp.max(cur, axis=-1, keepdims=True)
            idx = jnp.min(jnp.where(cur == m, iota, e), axis=-1, keepdims=True)
            vals.append(m)
            idxs.append(idx)
            cur = jnp.where(iota == idx, -jnp.inf, cur)
        vals = jnp.concatenate(vals, axis=-1)
        idxs = jnp.concatenate(idxs, axis=-1)
        w = jnp.exp(vals - vals[:, :1])
        w = w / jnp.sum(w, axis=-1, keepdims=True)
        idx_ref[...] = idxs
        wts_ref[...] = w


@functools.partial(jax.jit, static_argnames=("block_t",))
def _router(x, W1, b1, W2, b2, block_t=512):
    n, d = x.shape
    hdim = W1.shape[1]
    e = W2.shape[1]
    nblk = n // block_t
    last = nblk - 1
    grid = (nblk + 1,)
    out = pl.pallas_call(
        _router_block,
        grid=grid,
        in_specs=[
            pl.BlockSpec((block_t, d), lambda i: (jnp.minimum(i, last), 0)),
            pl.BlockSpec((d, hdim), lambda i: (0, 0)),
            pl.BlockSpec((1, hdim), lambda i: (0, 0)),
            pl.BlockSpec((hdim, e), lambda i: (0, 0)),
            pl.BlockSpec((1, e), lambda i: (0, 0)),
        ],
        out_specs=[
            pl.BlockSpec((block_t, _TOP_K), lambda i: (jnp.maximum(i - 1, 0), 0)),
            pl.BlockSpec((block_t, _TOP_K), lambda i: (jnp.maximum(i - 1, 0), 0)),
            pl.BlockSpec((block_t, e), lambda i: (jnp.minimum(i, last), 0)),
        ],
        out_shape=[
            jax.ShapeDtypeStruct((n, _TOP_K), jnp.int32),
            jax.ShapeDtypeStruct((n, _TOP_K), jnp.float32),
            jax.ShapeDtypeStruct((n, e), jnp.float32),
        ],
        scratch_shapes=[pltpu.VMEM((2, block_t, e), jnp.float32)],
    )(x, W1, b1.reshape(1, hdim), W2, b2.reshape(1, e))
    return out[0], out[1], out[2]


def kernel(x, W1, b1, W2, b2):
    return _router(x, W1, b1, W2, b2, block_t=1024)
